# Initial kernel scaffold; baseline (speedup 1.0000x reference)
import jax
import jax.numpy as jnp
from jax.experimental import pallas as pl


def kernel(input_ids, token_type_ids, word_table, pos_table, type_table, ln_gamma, ln_beta):
    raise NotImplementedError("write your pallas kernel here")


# speedup vs baseline: 3.9949x; 3.9949x over previous
import functools

import jax
import jax.numpy as jnp
from jax import lax
from jax.experimental import pallas as pl
from jax.experimental.pallas import tpu as pltpu
from jax.experimental.pallas import tpu_sc as plsc

_NC = 2
_NS = 16
_NW = _NC * _NS


def _sc_gather(flat_ids, word_table, chunk):
    n = flat_ids.shape[0]
    e = word_table.shape[1]
    per_w = n // _NW
    n_chunks = per_w // chunk

    mesh = plsc.VectorSubcoreMesh(
        core_axis_name="c", subcore_axis_name="s", num_cores=_NC, num_subcores=_NS
    )

    @functools.partial(
        pl.kernel,
        out_type=jax.ShapeDtypeStruct((n, e), jnp.float32),
        mesh=mesh,
        scratch_types=[
            pltpu.VMEM((chunk,), jnp.int32),
            pltpu.VMEM((chunk, e), jnp.float32),
            pltpu.SemaphoreType.DMA,
        ],
        compiler_params=pltpu.CompilerParams(use_tc_tiling_on_sc=False),
    )
    def gather_kernel(ids_hbm, table_hbm, out_hbm, idx_v, rows_v, sem):
        wid = lax.axis_index("s") * _NC + lax.axis_index("c")
        base = wid * per_w

        def body(i, carry):
            off = pl.multiple_of(base + i * chunk, 8)
            pltpu.sync_copy(ids_hbm.at[pl.ds(off, chunk)], idx_v)
            pltpu.async_copy(table_hbm.at[idx_v], rows_v, sem).wait()
            pltpu.sync_copy(rows_v, out_hbm.at[pl.ds(off, chunk)])
            return carry

        lax.fori_loop(0, n_chunks, body, 0)

    return gather_kernel(flat_ids, word_table)


def _tc_add_ln(word_emb, token_type_ids, pos_table, type_table, gamma, beta, bb):
    b, l = token_type_ids.shape
    e = pos_table.shape[1]

    def body(x_ref, tt_ref, pos_ref, type_ref, g_ref, b_ref, o_ref):
        x = x_ref[...]
        tt = tt_ref[...]
        pos = pos_ref[...]
        t0 = type_ref[0, :]
        dt = type_ref[1, :] - t0
        type_emb = t0[None, None, :] + tt * dt[None, None, :]
        emb = x + pos[None, :, :] + type_emb
        mean = jnp.mean(emb, axis=-1, keepdims=True)
        c = emb - mean
        var = jnp.mean(c * c, axis=-1, keepdims=True)
        inv = lax.rsqrt(var + 1e-5)
        o_ref[...] = c * inv * g_ref[...] + b_ref[...]

    return pl.pallas_call(
        body,
        grid=(b // bb,),
        in_specs=[
            pl.BlockSpec((bb, l, e), lambda i: (i, 0, 0)),
            pl.BlockSpec((bb, l, 1), lambda i: (i, 0, 0)),
            pl.BlockSpec((l, e), lambda i: (0, 0)),
            pl.BlockSpec((2, e), lambda i: (0, 0)),
            pl.BlockSpec((1, e), lambda i: (0, 0)),
            pl.BlockSpec((1, e), lambda i: (0, 0)),
        ],
        out_specs=pl.BlockSpec((bb, l, e), lambda i: (i, 0, 0)),
        out_shape=jax.ShapeDtypeStruct((b, l, e), jnp.float32),
    )(word_emb, token_type_ids.astype(jnp.float32).reshape(b, l, 1), pos_table,
      type_table, gamma.reshape(1, e), beta.reshape(1, e))


def kernel(input_ids, token_type_ids, word_table, pos_table, type_table, ln_gamma, ln_beta):
    b, l = input_ids.shape
    e = word_table.shape[1]
    flat_ids = input_ids.reshape(b * l)
    word_rows = _sc_gather(flat_ids, word_table, chunk=512)
    word_emb = word_rows.reshape(b, l, e)
    pos = pos_table[:l]
    return _tc_add_ln(word_emb, token_type_ids, pos, type_table, ln_gamma, ln_beta, bb=16)


# speedup vs baseline: 4.4430x; 1.1122x over previous
import functools

import jax
import jax.numpy as jnp
from jax import lax
from jax.experimental import pallas as pl
from jax.experimental.pallas import tpu as pltpu
from jax.experimental.pallas import tpu_sc as plsc

_NC = 2
_NS = 16
_NW = _NC * _NS


def _sc_gather_packed(flat_ids, word_table, chunk_rows):
    n = flat_ids.shape[0]
    e = word_table.shape[1]
    n2 = n // 2
    per_w = n2 // _NW
    n_chunks = per_w // chunk_rows

    mesh = plsc.VectorSubcoreMesh(
        core_axis_name="c", subcore_axis_name="s", num_cores=_NC, num_subcores=_NS
    )

    @functools.partial(
        pl.kernel,
        out_type=jax.ShapeDtypeStruct((n2, 2 * e), jnp.float32),
        mesh=mesh,
        scratch_types=[
            pltpu.VMEM((chunk_rows,), jnp.int32),
            pltpu.VMEM((chunk_rows,), jnp.int32),
            pltpu.VMEM((chunk_rows, e), jnp.float32),
            pltpu.VMEM((chunk_rows, e), jnp.float32),
            pltpu.SemaphoreType.DMA,
        ],
        compiler_params=pltpu.CompilerParams(use_tc_tiling_on_sc=False),
    )
    def gather_kernel(ids_hbm, table_hbm, out_hbm, idx_l, idx_r, lv, rv, sem):
        wid = lax.axis_index("s") * _NC + lax.axis_index("c")
        base = wid * per_w

        def body(i, carry):
            off = pl.multiple_of(base + i * chunk_rows, 8)
            pltpu.sync_copy(ids_hbm.at[pl.ds(off, chunk_rows)], idx_l)
            pltpu.sync_copy(ids_hbm.at[pl.ds(n2 + off, chunk_rows)], idx_r)
            cl = pltpu.async_copy(table_hbm.at[idx_l], lv, sem)
            cr = pltpu.async_copy(table_hbm.at[idx_r], rv, sem)
            cl.wait()
            cr.wait()
            pltpu.sync_copy(lv, out_hbm.at[pl.ds(off, chunk_rows), pl.ds(0, e)])
            pltpu.sync_copy(rv, out_hbm.at[pl.ds(off, chunk_rows), pl.ds(e, e)])
            return carry

        lax.fori_loop(0, n_chunks, body, 0)

    return gather_kernel(flat_ids, word_table)


def _tc_add_ln(xp, tt4, pos_table, type_table, gamma, beta, bb):
    _, b2, l, _ = tt4.shape
    e = pos_table.shape[1]
    rb = bb * l

    def body(x_ref, tt_ref, pos_ref, type_ref, g_ref, b_ref, o_ref):
        x = x_ref[...]
        pos = pos_ref[...]
        t0 = type_ref[0, :]
        dt = type_ref[1, :] - t0
        g = g_ref[...]
        bb_ = b_ref[...]
        for h in range(2):
            xh = x[:, h * e:(h + 1) * e].reshape(bb, l, e)
            tt = tt_ref[h]
            emb = xh + pos[None, :, :] + t0[None, None, :] + tt * dt[None, None, :]
            mean = jnp.mean(emb, axis=-1, keepdims=True)
            c = emb - mean
            var = jnp.mean(c * c, axis=-1, keepdims=True)
            inv = lax.rsqrt(var + 1e-5)
            o_ref[h] = c * inv * g + bb_

    return pl.pallas_call(
        body,
        grid=(b2 // bb,),
        in_specs=[
            pl.BlockSpec((rb, 2 * e), lambda i: (i, 0)),
            pl.BlockSpec((2, bb, l, 1), lambda i: (0, i, 0, 0)),
            pl.BlockSpec((l, e), lambda i: (0, 0)),
            pl.BlockSpec((2, e), lambda i: (0, 0)),
            pl.BlockSpec((1, e), lambda i: (0, 0)),
            pl.BlockSpec((1, e), lambda i: (0, 0)),
        ],
        out_specs=pl.BlockSpec((2, bb, l, e), lambda i: (0, i, 0, 0)),
        out_shape=jax.ShapeDtypeStruct((2, b2, l, e), jnp.float32),
    )(xp, tt4, pos_table, type_table, gamma.reshape(1, e), beta.reshape(1, e))


def kernel(input_ids, token_type_ids, word_table, pos_table, type_table, ln_gamma, ln_beta):
    b, l = input_ids.shape
    e = word_table.shape[1]
    flat_ids = input_ids.reshape(b * l)
    xp = _sc_gather_packed(flat_ids, word_table, chunk_rows=256)
    tt4 = token_type_ids.astype(jnp.float32).reshape(2, b // 2, l, 1)
    out = _tc_add_ln(xp, tt4, pos_table[:l], type_table, ln_gamma, ln_beta, bb=4)
    return out.reshape(b, l, e)
